# baseline (speedup 1.0000x reference)
import jax
import jax.numpy as jnp
from jax.experimental import pallas as pl
from jax.experimental.pallas import tpu as pltpu


def _gates_body(x_ref, wg_ref, bg_ref, out_ref):
    j = pl.program_id(1)
    g = jnp.dot(x_ref[...], wg_ref[...], preferred_element_type=jnp.float32)
    sg = jax.nn.sigmoid(g + bg_ref[...])
    tn, c = sg.shape
    part = jnp.sum(sg.reshape(tn // 8, 8, c), axis=0)

    @pl.when(j == 0)
    def _():
        out_ref[...] = jnp.zeros_like(out_ref)

    out_ref[0] += part


def _grad_body(x_ref, wkv_hbm, m1t_hbm, m2t_hbm, g1t_hbm, g2t_hbm,
               wkv, m1t, m2t, g1t_ref, g2t_ref, sems):
    i = pl.program_id(0)
    j = pl.program_id(1)
    nb = pl.num_programs(1)
    d = x_ref.shape[1]

    @pl.when(j == 0)
    def _():
        c0 = pltpu.make_async_copy(wkv_hbm, wkv, sems.at[0])
        c1 = pltpu.make_async_copy(m1t_hbm, m1t, sems.at[1])
        c2 = pltpu.make_async_copy(m2t_hbm, m2t, sems.at[2])
        c0.start(); c1.start(); c2.start()
        c0.wait(); c1.wait(); c2.wait()
        g1t_ref[...] = jnp.zeros_like(g1t_ref)
        g2t_ref[...] = jnp.zeros_like(g2t_ref)

    kv = jnp.dot(x_ref[...], wkv[...], preferred_element_type=jnp.float32)
    k = kv[:, :d]
    v = kv[:, d:]
    h = jnp.dot(k, m1t[...], preferred_element_type=jnp.float32)
    sig = jax.nn.sigmoid(h)
    a = h * sig
    pred = jnp.dot(a, m2t[...], preferred_element_type=jnp.float32)
    r = (pred - v) * (2.0 / d)
    da = jax.lax.dot_general(r, m2t[...], (((1,), (1,)), ((), ())),
                             preferred_element_type=jnp.float32)
    dh = da * (sig * (1.0 + h * (1.0 - sig)))
    g1t_ref[...] += jax.lax.dot_general(k, dh, (((0,), (0,)), ((), ())),
                                        preferred_element_type=jnp.float32)
    g2t_ref[...] += jax.lax.dot_general(a, r, (((0,), (0,)), ((), ())),
                                        preferred_element_type=jnp.float32)

    @pl.when(j == nb - 1)
    def _():
        c3 = pltpu.make_async_copy(g1t_ref, g1t_hbm.at[i], sems.at[0])
        c4 = pltpu.make_async_copy(g2t_ref, g2t_hbm.at[i], sems.at[1])
        c3.start(); c4.start()
        c3.wait(); c4.wait()


def _update_body(sc_ref, mt_ref, st_ref, ga_ref, gb_ref, out_ref):
    alpha = sc_ref[0]
    theta = sc_ref[1]
    eta = sc_ref[2]
    out_ref[...] = ((1.0 - alpha) * mt_ref[...] + eta * st_ref[...]
                    - theta * (ga_ref[...] + gb_ref[...]))


def _retr_body(x_ref, wq_ref, wout_ref, m1nt_hbm, m2nt_hbm, out_ref,
               m1nt, m2nt, sems):
    j = pl.program_id(1)

    @pl.when(j == 0)
    def _():
        c0 = pltpu.make_async_copy(m1nt_hbm, m1nt, sems.at[0])
        c1 = pltpu.make_async_copy(m2nt_hbm, m2nt, sems.at[1])
        c0.start(); c1.start()
        c0.wait(); c1.wait()

    q = jnp.dot(x_ref[...], wq_ref[...], preferred_element_type=jnp.float32)
    hq = jnp.dot(q, m1nt[...], preferred_element_type=jnp.float32)
    aq = hq * jax.nn.sigmoid(hq)
    retr = jnp.dot(aq, m2nt[...], preferred_element_type=jnp.float32)
    out_ref[...] = jnp.dot(retr, wout_ref[...],
                           preferred_element_type=jnp.float32)


def kernel(x, Wk, Wv, Wq, Wout, Wgd, bgd, Wgl, bgl, Wgm, bgm, M1, M2, S1, S2):
    b, s, d = x.shape
    h = M1.shape[0]
    n = b * s
    xf = x.reshape(n, d)
    f32 = jnp.float32

    ncores = 2
    vmem = pltpu.CompilerParams(
        dimension_semantics=("parallel", "arbitrary"),
        vmem_limit_bytes=56 * 1024 * 1024,
    )

    wg = jnp.concatenate([Wgd.T, Wgl.T, Wgm.T], axis=1)
    bg = jnp.concatenate([bgd, bgl, bgm]).reshape(1, 3 * d)
    tng = min(512, n // ncores)
    nbg = n // (ncores * tng)
    gate_sums = pl.pallas_call(
        _gates_body,
        grid=(ncores, nbg),
        in_specs=[
            pl.BlockSpec((tng, d), lambda i, j: (i * nbg + j, 0)),
            pl.BlockSpec((d, 3 * d), lambda i, j: (0, 0)),
            pl.BlockSpec((1, 3 * d), lambda i, j: (0, 0)),
        ],
        out_specs=pl.BlockSpec((1, 8, 3 * d), lambda i, j: (i, 0, 0)),
        out_shape=jax.ShapeDtypeStruct((ncores, 8, 3 * d), f32),
        compiler_params=vmem,
        name="ltm_gates",
    )(xf, wg, bg)

    alpha = jnp.sum(gate_sums[:, :, :d]) / (n * d)
    theta = jnp.sum(gate_sums[:, :, d:2 * d]) / (n * d)
    eta = jnp.sum(gate_sums[:, :, 2 * d:]) / (n * d)
    scalars = jnp.stack([alpha, theta, eta])

    wkv = jnp.concatenate([Wk.T, Wv.T], axis=1)
    m1t = M1.T
    m2t = M2.T
    tn = min(256, n // ncores)
    nb = n // (ncores * tn)
    g1tp, g2tp = pl.pallas_call(
        _grad_body,
        grid=(ncores, nb),
        in_specs=[
            pl.BlockSpec((tn, d), lambda i, j: (i * nb + j, 0)),
            pl.BlockSpec(memory_space=pl.ANY),
            pl.BlockSpec(memory_space=pl.ANY),
            pl.BlockSpec(memory_space=pl.ANY),
        ],
        out_specs=[
            pl.BlockSpec(memory_space=pl.ANY),
            pl.BlockSpec(memory_space=pl.ANY),
        ],
        out_shape=[
            jax.ShapeDtypeStruct((ncores, d, h), f32),
            jax.ShapeDtypeStruct((ncores, h, d), f32),
        ],
        scratch_shapes=[
            pltpu.VMEM((d, 2 * d), f32),
            pltpu.VMEM((d, h), f32),
            pltpu.VMEM((h, d), f32),
            pltpu.VMEM((d, h), f32),
            pltpu.VMEM((h, d), f32),
            pltpu.SemaphoreType.DMA((3,)),
        ],
        compiler_params=vmem,
        name="ltm_grad",
    )(xf, wkv, m1t, m2t)

    def _update(mt, st, gp, rows, cols):
        rb = 8
        return pl.pallas_call(
            _update_body,
            grid=(rb,),
            in_specs=[
                pl.BlockSpec(memory_space=pltpu.SMEM),
                pl.BlockSpec((rows // rb, cols), lambda i: (i, 0)),
                pl.BlockSpec((rows // rb, cols), lambda i: (i, 0)),
                pl.BlockSpec((rows // rb, cols), lambda i: (i, 0)),
                pl.BlockSpec((rows // rb, cols), lambda i: (i, 0)),
            ],
            out_specs=pl.BlockSpec((rows // rb, cols), lambda i: (i, 0)),
            out_shape=jax.ShapeDtypeStruct((rows, cols), f32),
            compiler_params=pltpu.CompilerParams(
                dimension_semantics=("parallel",),
            ),
            name="ltm_update",
        )(scalars, mt, st, gp[0], gp[1])

    m1nt = _update(m1t, S1.T, g1tp, d, h)
    m2nt = _update(m2t, S2.T, g2tp, h, d)

    wqt = Wq.T
    woutt = Wout.T
    tnr = min(256, n // ncores)
    nbr = n // (ncores * tnr)
    out = pl.pallas_call(
        _retr_body,
        grid=(ncores, nbr),
        in_specs=[
            pl.BlockSpec((tnr, d), lambda i, j: (i * nbr + j, 0)),
            pl.BlockSpec((d, d), lambda i, j: (0, 0)),
            pl.BlockSpec((d, d), lambda i, j: (0, 0)),
            pl.BlockSpec(memory_space=pl.ANY),
            pl.BlockSpec(memory_space=pl.ANY),
        ],
        out_specs=pl.BlockSpec((tnr, d), lambda i, j: (i * nbr + j, 0)),
        out_shape=jax.ShapeDtypeStruct((n, d), f32),
        scratch_shapes=[
            pltpu.VMEM((d, h), f32),
            pltpu.VMEM((h, d), f32),
            pltpu.SemaphoreType.DMA((2,)),
        ],
        compiler_params=vmem,
        name="ltm_retrieve",
    )(xf, wqt, woutt, m1nt, m2nt)

    return out.reshape(b, s, d)


# speedup vs baseline: 1.0529x; 1.0529x over previous
import jax
import jax.numpy as jnp
from jax.experimental import pallas as pl
from jax.experimental.pallas import tpu as pltpu

_BF = jnp.bfloat16
_F32 = jnp.float32
_TN = (((0,), (0,)), ((), ()))


def _gates_body(x_ref, wg_ref, bg_ref, out_ref):
    j = pl.program_id(1)
    g = jnp.dot(x_ref[...], wg_ref[...], preferred_element_type=_F32)
    t = jnp.tanh(0.5 * (g + bg_ref[...]))
    tn, c = t.shape
    part = jnp.sum(t.reshape(8, tn // 8, c), axis=0)

    @pl.when(j == 0)
    def _():
        out_ref[...] = jnp.zeros_like(out_ref)

    out_ref[0] += part


def _grad_body(x_ref, wkvt_hbm, m1t_hbm, m2t_hbm, m2_hbm, g1_hbm, g2_hbm,
               wkvt, m1t, m2t, m2, ka, aa, ra, dha, g1_ref, g2_ref, sems):
    i = pl.program_id(0)
    j = pl.program_id(1)
    nb = pl.num_programs(1)
    d = x_ref.shape[1]
    tn = x_ref.shape[0]
    sub = tn // 2

    @pl.when(j == 0)
    def _():
        c0 = pltpu.make_async_copy(wkvt_hbm, wkvt, sems.at[0])
        c1 = pltpu.make_async_copy(m1t_hbm, m1t, sems.at[1])
        c2 = pltpu.make_async_copy(m2t_hbm, m2t, sems.at[2])
        c3 = pltpu.make_async_copy(m2_hbm, m2, sems.at[3])
        c0.start(); c1.start(); c2.start(); c3.start()
        c0.wait(); c1.wait(); c2.wait(); c3.wait()
        g1_ref[...] = jnp.zeros_like(g1_ref)
        g2_ref[...] = jnp.zeros_like(g2_ref)

    for p in range(2):
        sl = slice(p * sub, (p + 1) * sub)
        kv = jnp.dot(x_ref[sl, :], wkvt[...], preferred_element_type=_F32)
        k = kv[:, :d].astype(_BF)
        v = kv[:, d:]
        ka[sl, :] = k
        h = jnp.dot(k, m1t[...], preferred_element_type=_F32)
        sig = 0.5 * (1.0 + jnp.tanh(0.5 * h))
        a = h * sig
        a_bf = a.astype(_BF)
        aa[sl, :] = a_bf
        pred = jnp.dot(a_bf, m2t[...], preferred_element_type=_F32)
        r_bf = ((pred - v) * (2.0 / d)).astype(_BF)
        ra[sl, :] = r_bf
        da = jnp.dot(r_bf, m2[...], preferred_element_type=_F32)
        dha[sl, :] = (da * (sig * (1.0 + h * (1.0 - sig)))).astype(_BF)

    g1_ref[...] += jax.lax.dot_general(dha[...], ka[...], _TN,
                                       preferred_element_type=_F32)
    g2_ref[...] += jax.lax.dot_general(ra[...], aa[...], _TN,
                                       preferred_element_type=_F32)

    @pl.when(j == nb - 1)
    def _():
        c4 = pltpu.make_async_copy(g1_ref, g1_hbm.at[i], sems.at[0])
        c5 = pltpu.make_async_copy(g2_ref, g2_hbm.at[i], sems.at[1])
        c4.start(); c5.start()
        c4.wait(); c5.wait()


def _update_body(sc_ref, m_ref, s_ref, ga_ref, gb_ref, out_ref):
    alpha = sc_ref[0]
    theta = sc_ref[1]
    eta = sc_ref[2]
    upd = ((1.0 - alpha) * m_ref[...] + eta * s_ref[...]
           - theta * (ga_ref[...] + gb_ref[...]))
    out_ref[...] = upd.astype(_BF)


def _retr_body(x_ref, wqt_ref, woutt_ref, m1nt_hbm, m2nt_hbm, out_ref,
               m1nt, m2nt, sems):
    j = pl.program_id(1)

    @pl.when(j == 0)
    def _():
        c0 = pltpu.make_async_copy(m1nt_hbm, m1nt, sems.at[0])
        c1 = pltpu.make_async_copy(m2nt_hbm, m2nt, sems.at[1])
        c0.start(); c1.start()
        c0.wait(); c1.wait()

    half = x_ref.shape[0] // 2
    for p in range(2):
        sl = slice(p * half, (p + 1) * half)
        q = jnp.dot(x_ref[sl, :], wqt_ref[...],
                    preferred_element_type=_F32).astype(_BF)
        hq = jnp.dot(q, m1nt[...], preferred_element_type=_F32)
        aq = (hq * (0.5 * (1.0 + jnp.tanh(0.5 * hq)))).astype(_BF)
        retr = jnp.dot(aq, m2nt[...],
                       preferred_element_type=_F32).astype(_BF)
        out_ref[sl, :] = jnp.dot(retr, woutt_ref[...],
                                 preferred_element_type=_F32)


def kernel(x, Wk, Wv, Wq, Wout, Wgd, bgd, Wgl, bgl, Wgm, bgm, M1, M2, S1, S2):
    b, s, d = x.shape
    h = M1.shape[0]
    n = b * s
    xf = x.reshape(n, d).astype(_BF)

    ncores = 2
    vmem = pltpu.CompilerParams(
        dimension_semantics=("parallel", "arbitrary"),
        vmem_limit_bytes=56 * 1024 * 1024,
    )

    wgt = jnp.concatenate([Wgd, Wgl, Wgm], axis=0).astype(_BF).T
    bg = jnp.concatenate([bgd, bgl, bgm]).reshape(1, 3 * d)
    tng = min(1024, n // ncores)
    nbg = n // (ncores * tng)
    gate_sums = pl.pallas_call(
        _gates_body,
        grid=(ncores, nbg),
        in_specs=[
            pl.BlockSpec((tng, d), lambda i, j: (i * nbg + j, 0)),
            pl.BlockSpec((d, 3 * d), lambda i, j: (0, 0)),
            pl.BlockSpec((1, 3 * d), lambda i, j: (0, 0)),
        ],
        out_specs=pl.BlockSpec((1, tng // 8, 3 * d), lambda i, j: (i, 0, 0)),
        out_shape=jax.ShapeDtypeStruct((ncores, tng // 8, 3 * d), _F32),
        compiler_params=vmem,
        name="ltm_gates",
    )(xf, wgt, bg)

    alpha = 0.5 + 0.5 * jnp.sum(gate_sums[:, :, :d]) / (n * d)
    theta = 0.5 + 0.5 * jnp.sum(gate_sums[:, :, d:2 * d]) / (n * d)
    eta = 0.5 + 0.5 * jnp.sum(gate_sums[:, :, 2 * d:]) / (n * d)
    scalars = jnp.stack([alpha, theta, eta])

    wkvt = jnp.concatenate([Wk, Wv], axis=0).astype(_BF).T
    m1t_bf = M1.astype(_BF).T
    m2t_bf = M2.astype(_BF).T
    m2_bf = M2.astype(_BF)
    tn = min(512, n // ncores)
    nb = n // (ncores * tn)
    g1p, g2p = pl.pallas_call(
        _grad_body,
        grid=(ncores, nb),
        in_specs=[
            pl.BlockSpec((tn, d), lambda i, j: (i * nb + j, 0)),
            pl.BlockSpec(memory_space=pl.ANY),
            pl.BlockSpec(memory_space=pl.ANY),
            pl.BlockSpec(memory_space=pl.ANY),
            pl.BlockSpec(memory_space=pl.ANY),
        ],
        out_specs=[
            pl.BlockSpec(memory_space=pl.ANY),
            pl.BlockSpec(memory_space=pl.ANY),
        ],
        out_shape=[
            jax.ShapeDtypeStruct((ncores, h, d), _F32),
            jax.ShapeDtypeStruct((ncores, d, h), _F32),
        ],
        scratch_shapes=[
            pltpu.VMEM((d, 2 * d), _BF),
            pltpu.VMEM((d, h), _BF),
            pltpu.VMEM((h, d), _BF),
            pltpu.VMEM((d, h), _BF),
            pltpu.VMEM((tn, d), _BF),
            pltpu.VMEM((tn, h), _BF),
            pltpu.VMEM((tn, d), _BF),
            pltpu.VMEM((tn, h), _BF),
            pltpu.VMEM((h, d), _F32),
            pltpu.VMEM((d, h), _F32),
            pltpu.SemaphoreType.DMA((4,)),
        ],
        compiler_params=vmem,
        name="ltm_grad",
    )(xf, wkvt, m1t_bf, m2t_bf, m2_bf)

    def _update(m, st, gp, rows, cols):
        rb = 8
        return pl.pallas_call(
            _update_body,
            grid=(rb,),
            in_specs=[
                pl.BlockSpec(memory_space=pltpu.SMEM),
                pl.BlockSpec((rows // rb, cols), lambda i: (i, 0)),
                pl.BlockSpec((rows // rb, cols), lambda i: (i, 0)),
                pl.BlockSpec((rows // rb, cols), lambda i: (i, 0)),
                pl.BlockSpec((rows // rb, cols), lambda i: (i, 0)),
            ],
            out_specs=pl.BlockSpec((rows // rb, cols), lambda i: (i, 0)),
            out_shape=jax.ShapeDtypeStruct((rows, cols), _BF),
            compiler_params=pltpu.CompilerParams(
                dimension_semantics=("parallel",),
            ),
            name="ltm_update",
        )(scalars, m, st, gp[0], gp[1])

    m1n = _update(M1, S1, g1p, h, d)
    m2n = _update(M2, S2, g2p, d, h)

    wqt = Wq.astype(_BF).T
    woutt = Wout.astype(_BF).T
    m1nt = m1n.T
    m2nt = m2n.T
    tnr = min(1024, n // ncores)
    nbr = n // (ncores * tnr)
    out = pl.pallas_call(
        _retr_body,
        grid=(ncores, nbr),
        in_specs=[
            pl.BlockSpec((tnr, d), lambda i, j: (i * nbr + j, 0)),
            pl.BlockSpec((d, d), lambda i, j: (0, 0)),
            pl.BlockSpec((d, d), lambda i, j: (0, 0)),
            pl.BlockSpec(memory_space=pl.ANY),
            pl.BlockSpec(memory_space=pl.ANY),
        ],
        out_specs=pl.BlockSpec((tnr, d), lambda i, j: (i * nbr + j, 0)),
        out_shape=jax.ShapeDtypeStruct((n, d), _F32),
        scratch_shapes=[
            pltpu.VMEM((d, h), _BF),
            pltpu.VMEM((h, d), _BF),
            pltpu.SemaphoreType.DMA((2,)),
        ],
        compiler_params=vmem,
        name="ltm_retrieve",
    )(xf, wqt, woutt, m1nt, m2nt)

    return out.reshape(b, s, d)


# speedup vs baseline: 1.0929x; 1.0380x over previous
import jax
import jax.numpy as jnp
from jax.experimental import pallas as pl
from jax.experimental.pallas import tpu as pltpu

_BF = jnp.bfloat16
_F32 = jnp.float32
_TN = (((0,), (0,)), ((), ()))


def _gradg_body(x_ref, wkvt_hbm, m1t_hbm, m2t_hbm, m2_hbm, wgt_hbm, hbg_ref,
                gs_ref, g1_hbm, g2_hbm,
                wkvt, m1t, m2t, m2, wgt, ka, aa, ra, dha,
                g1_ref, g2_ref, sems):
    i = pl.program_id(0)
    j = pl.program_id(1)
    nb = pl.num_programs(1)
    d = x_ref.shape[1]
    tn = x_ref.shape[0]
    sub = tn // 2

    @pl.when(j == 0)
    def _():
        c0 = pltpu.make_async_copy(wkvt_hbm, wkvt, sems.at[0])
        c1 = pltpu.make_async_copy(m1t_hbm, m1t, sems.at[1])
        c2 = pltpu.make_async_copy(m2t_hbm, m2t, sems.at[2])
        c3 = pltpu.make_async_copy(m2_hbm, m2, sems.at[3])
        c4 = pltpu.make_async_copy(wgt_hbm, wgt, sems.at[4])
        c0.start(); c1.start(); c2.start(); c3.start(); c4.start()
        c0.wait(); c1.wait(); c2.wait(); c3.wait(); c4.wait()
        g1_ref[...] = jnp.zeros_like(g1_ref)
        g2_ref[...] = jnp.zeros_like(g2_ref)
        gs_ref[...] = jnp.zeros_like(gs_ref)

    for p in range(2):
        sl = slice(p * sub, (p + 1) * sub)
        xs = x_ref[sl, :].astype(_BF)
        gg = jnp.dot(xs, wgt[...], preferred_element_type=_F32)
        t = jnp.tanh(0.5 * gg + hbg_ref[...])
        c = t.shape[1]
        gs_ref[0] += jnp.sum(t.reshape(sub // 8, 8, c), axis=0)

        kv = jnp.dot(xs, wkvt[...], preferred_element_type=_F32)
        k = kv[:, :d].astype(_BF)
        v = kv[:, d:]
        ka[sl, :] = k
        h = jnp.dot(k, m1t[...], preferred_element_type=_F32)
        sig = 0.5 * (1.0 + jnp.tanh(0.5 * h))
        a = h * sig
        a_bf = a.astype(_BF)
        aa[sl, :] = a_bf
        pred = jnp.dot(a_bf, m2t[...], preferred_element_type=_F32)
        r_bf = ((pred - v) * (2.0 / d)).astype(_BF)
        ra[sl, :] = r_bf
        da = jnp.dot(r_bf, m2[...], preferred_element_type=_F32)
        dha[sl, :] = (da * (sig * (1.0 + h * (1.0 - sig)))).astype(_BF)

    g1_ref[...] += jax.lax.dot_general(dha[...], ka[...], _TN,
                                       preferred_element_type=_F32)
    g2_ref[...] += jax.lax.dot_general(ra[...], aa[...], _TN,
                                       preferred_element_type=_F32)

    @pl.when(j == nb - 1)
    def _():
        c5 = pltpu.make_async_copy(g1_ref, g1_hbm.at[i], sems.at[0])
        c6 = pltpu.make_async_copy(g2_ref, g2_hbm.at[i], sems.at[1])
        c5.start(); c6.start()
        c5.wait(); c6.wait()


def _update_body(sc_ref, m_ref, s_ref, ga_ref, gb_ref, out_ref):
    alpha = sc_ref[0]
    theta = sc_ref[1]
    eta = sc_ref[2]
    upd = ((1.0 - alpha) * m_ref[...] + eta * s_ref[...]
           - theta * (ga_ref[...] + gb_ref[...]))
    out_ref[...] = upd.astype(_BF)


def _retr_body(x_ref, wqt_ref, woutt_ref, m1nt_hbm, m2nt_hbm, out_ref,
               m1nt, m2nt, sems):
    j = pl.program_id(1)

    @pl.when(j == 0)
    def _():
        c0 = pltpu.make_async_copy(m1nt_hbm, m1nt, sems.at[0])
        c1 = pltpu.make_async_copy(m2nt_hbm, m2nt, sems.at[1])
        c0.start(); c1.start()
        c0.wait(); c1.wait()

    half = x_ref.shape[0] // 2
    for p in range(2):
        sl = slice(p * half, (p + 1) * half)
        q = jnp.dot(x_ref[sl, :].astype(_BF), wqt_ref[...],
                    preferred_element_type=_F32).astype(_BF)
        hq = jnp.dot(q, m1nt[...], preferred_element_type=_F32)
        aq = (hq * (0.5 * (1.0 + jnp.tanh(0.5 * hq)))).astype(_BF)
        retr = jnp.dot(aq, m2nt[...],
                       preferred_element_type=_F32).astype(_BF)
        out_ref[sl, :] = jnp.dot(retr, woutt_ref[...],
                                 preferred_element_type=_F32)


def kernel(x, Wk, Wv, Wq, Wout, Wgd, bgd, Wgl, bgl, Wgm, bgm, M1, M2, S1, S2):
    b, s, d = x.shape
    h = M1.shape[0]
    n = b * s
    xf = x.reshape(n, d)

    ncores = 2
    vmem = pltpu.CompilerParams(
        dimension_semantics=("parallel", "arbitrary"),
        vmem_limit_bytes=58 * 1024 * 1024,
    )

    wgt = jnp.concatenate([Wgd, Wgl, Wgm], axis=0).astype(_BF).T
    hbg = 0.5 * jnp.concatenate([bgd, bgl, bgm]).reshape(1, 3 * d)
    wkvt = jnp.concatenate([Wk, Wv], axis=0).astype(_BF).T
    m1t_bf = M1.astype(_BF).T
    m2t_bf = M2.astype(_BF).T
    m2_bf = M2.astype(_BF)

    tn = min(512, n // ncores)
    nb = n // (ncores * tn)
    gate_sums, g1p, g2p = pl.pallas_call(
        _gradg_body,
        grid=(ncores, nb),
        in_specs=[
            pl.BlockSpec((tn, d), lambda i, j: (i * nb + j, 0)),
            pl.BlockSpec(memory_space=pl.ANY),
            pl.BlockSpec(memory_space=pl.ANY),
            pl.BlockSpec(memory_space=pl.ANY),
            pl.BlockSpec(memory_space=pl.ANY),
            pl.BlockSpec(memory_space=pl.ANY),
            pl.BlockSpec((1, 3 * d), lambda i, j: (0, 0)),
        ],
        out_specs=[
            pl.BlockSpec((1, 8, 3 * d), lambda i, j: (i, 0, 0)),
            pl.BlockSpec(memory_space=pl.ANY),
            pl.BlockSpec(memory_space=pl.ANY),
        ],
        out_shape=[
            jax.ShapeDtypeStruct((ncores, 8, 3 * d), _F32),
            jax.ShapeDtypeStruct((ncores, h, d), _F32),
            jax.ShapeDtypeStruct((ncores, d, h), _F32),
        ],
        scratch_shapes=[
            pltpu.VMEM((d, 2 * d), _BF),
            pltpu.VMEM((d, h), _BF),
            pltpu.VMEM((h, d), _BF),
            pltpu.VMEM((d, h), _BF),
            pltpu.VMEM((d, 3 * d), _BF),
            pltpu.VMEM((tn, d), _BF),
            pltpu.VMEM((tn, h), _BF),
            pltpu.VMEM((tn, d), _BF),
            pltpu.VMEM((tn, h), _BF),
            pltpu.VMEM((h, d), _F32),
            pltpu.VMEM((d, h), _F32),
            pltpu.SemaphoreType.DMA((5,)),
        ],
        compiler_params=vmem,
        name="ltm_gradg",
    )(xf, wkvt, m1t_bf, m2t_bf, m2_bf, wgt, hbg)

    alpha = 0.5 + 0.5 * jnp.sum(gate_sums[:, :, :d]) / (n * d)
    theta = 0.5 + 0.5 * jnp.sum(gate_sums[:, :, d:2 * d]) / (n * d)
    eta = 0.5 + 0.5 * jnp.sum(gate_sums[:, :, 2 * d:]) / (n * d)
    scalars = jnp.stack([alpha, theta, eta])

    def _update(m, st, gp, rows, cols):
        rb = 8
        return pl.pallas_call(
            _update_body,
            grid=(rb,),
            in_specs=[
                pl.BlockSpec(memory_space=pltpu.SMEM),
                pl.BlockSpec((rows // rb, cols), lambda i: (i, 0)),
                pl.BlockSpec((rows // rb, cols), lambda i: (i, 0)),
                pl.BlockSpec((rows // rb, cols), lambda i: (i, 0)),
                pl.BlockSpec((rows // rb, cols), lambda i: (i, 0)),
            ],
            out_specs=pl.BlockSpec((rows // rb, cols), lambda i: (i, 0)),
            out_shape=jax.ShapeDtypeStruct((rows, cols), _BF),
            compiler_params=pltpu.CompilerParams(
                dimension_semantics=("parallel",),
            ),
            name="ltm_update",
        )(scalars, m, st, gp[0], gp[1])

    m1n = _update(M1, S1, g1p, h, d)
    m2n = _update(M2, S2, g2p, d, h)

    wqt = Wq.astype(_BF).T
    woutt = Wout.astype(_BF).T
    m1nt = m1n.T
    m2nt = m2n.T
    tnr = min(1024, n // ncores)
    nbr = n // (ncores * tnr)
    out = pl.pallas_call(
        _retr_body,
        grid=(ncores, nbr),
        in_specs=[
            pl.BlockSpec((tnr, d), lambda i, j: (i * nbr + j, 0)),
            pl.BlockSpec((d, d), lambda i, j: (0, 0)),
            pl.BlockSpec((d, d), lambda i, j: (0, 0)),
            pl.BlockSpec(memory_space=pl.ANY),
            pl.BlockSpec(memory_space=pl.ANY),
        ],
        out_specs=pl.BlockSpec((tnr, d), lambda i, j: (i * nbr + j, 0)),
        out_shape=jax.ShapeDtypeStruct((n, d), _F32),
        scratch_shapes=[
            pltpu.VMEM((d, h), _BF),
            pltpu.VMEM((h, d), _BF),
            pltpu.SemaphoreType.DMA((2,)),
        ],
        compiler_params=vmem,
        name="ltm_retrieve",
    )(xf, wqt, woutt, m1nt, m2nt)

    return out.reshape(b, s, d)


# speedup vs baseline: 1.1115x; 1.0170x over previous
import jax
import jax.numpy as jnp
from jax.experimental import pallas as pl
from jax.experimental.pallas import tpu as pltpu

_BF = jnp.bfloat16
_F32 = jnp.float32
_TN = (((0,), (0,)), ((), ()))
_NT = (((1,), (1,)), ((), ()))


def _gradg_body(x_ref, wkvt_hbm, m1t_hbm, m2t_hbm, m2_hbm, wgt_hbm, hbg_ref,
                gs_ref, g1_hbm, g2_hbm,
                wkvt, m1t, m2t, m2, wgt, ka, aa, ra, dha,
                g1_ref, g2_ref, sems):
    i = pl.program_id(0)
    j = pl.program_id(1)
    nb = pl.num_programs(1)
    d = x_ref.shape[1]
    tn = x_ref.shape[0]
    sub = tn // 2

    @pl.when(j == 0)
    def _():
        c0 = pltpu.make_async_copy(wkvt_hbm, wkvt, sems.at[0])
        c1 = pltpu.make_async_copy(m1t_hbm, m1t, sems.at[1])
        c2 = pltpu.make_async_copy(m2t_hbm, m2t, sems.at[2])
        c3 = pltpu.make_async_copy(m2_hbm, m2, sems.at[3])
        c4 = pltpu.make_async_copy(wgt_hbm, wgt, sems.at[4])
        c0.start(); c1.start(); c2.start(); c3.start(); c4.start()
        c0.wait(); c1.wait(); c2.wait(); c3.wait(); c4.wait()
        g1_ref[...] = jnp.zeros_like(g1_ref)
        g2_ref[...] = jnp.zeros_like(g2_ref)
        gs_ref[...] = jnp.zeros_like(gs_ref)

    for p in range(2):
        sl = slice(p * sub, (p + 1) * sub)
        xs = x_ref[sl, :].astype(_BF)
        gg = jnp.dot(xs, wgt[...], preferred_element_type=_F32)
        t = jnp.tanh(0.5 * gg + hbg_ref[...])
        c = t.shape[1]
        gs_ref[0] += jnp.sum(t.reshape(sub // 8, 8, c), axis=0)

        kv = jnp.dot(xs, wkvt[...], preferred_element_type=_F32)
        k = kv[:, :d].astype(_BF)
        v = kv[:, d:]
        ka[sl, :] = k
        h = jnp.dot(k, m1t[...], preferred_element_type=_F32)
        sig = 0.5 * (1.0 + jnp.tanh(0.5 * h))
        a = h * sig
        a_bf = a.astype(_BF)
        aa[sl, :] = a_bf
        pred = jnp.dot(a_bf, m2t[...], preferred_element_type=_F32)
        r_bf = ((pred - v) * (2.0 / d)).astype(_BF)
        ra[sl, :] = r_bf
        da = jnp.dot(r_bf, m2[...], preferred_element_type=_F32)
        dha[sl, :] = (da * (sig * (1.0 + h * (1.0 - sig)))).astype(_BF)

    g1_ref[...] += jax.lax.dot_general(dha[...], ka[...], _TN,
                                       preferred_element_type=_F32)
    g2_ref[...] += jax.lax.dot_general(ra[...], aa[...], _TN,
                                       preferred_element_type=_F32)

    @pl.when(j == nb - 1)
    def _():
        c5 = pltpu.make_async_copy(g1_ref, g1_hbm.at[i], sems.at[0])
        c6 = pltpu.make_async_copy(g2_ref, g2_hbm.at[i], sems.at[1])
        c5.start(); c6.start()
        c5.wait(); c6.wait()


def _update_body(sc_ref, m_ref, s_ref, ga_ref, gb_ref, out_ref):
    alpha = sc_ref[0]
    theta = sc_ref[1]
    eta = sc_ref[2]
    upd = ((1.0 - alpha) * m_ref[...] + eta * s_ref[...]
           - theta * (ga_ref[...] + gb_ref[...]))
    out_ref[...] = upd.astype(_BF)


def _retr_body(x_ref, wqt_ref, woutt_ref, m1nt_hbm, m2nt_hbm, out_ref,
               m1nt, m2nt, sems):
    j = pl.program_id(1)

    @pl.when(j == 0)
    def _():
        c0 = pltpu.make_async_copy(m1nt_hbm, m1nt, sems.at[0])
        c1 = pltpu.make_async_copy(m2nt_hbm, m2nt, sems.at[1])
        c0.start(); c1.start()
        c0.wait(); c1.wait()

    half = x_ref.shape[0] // 2
    for p in range(2):
        sl = slice(p * half, (p + 1) * half)
        q = jax.lax.dot_general(x_ref[sl, :].astype(_BF), wqt_ref[...], _NT,
                                preferred_element_type=_F32).astype(_BF)
        hq = jax.lax.dot_general(q, m1nt[...], _NT,
                                 preferred_element_type=_F32)
        aq = (hq * (0.5 * (1.0 + jnp.tanh(0.5 * hq)))).astype(_BF)
        retr = jax.lax.dot_general(aq, m2nt[...], _NT,
                                   preferred_element_type=_F32).astype(_BF)
        out_ref[sl, :] = jax.lax.dot_general(retr, woutt_ref[...], _NT,
                                             preferred_element_type=_F32)


def kernel(x, Wk, Wv, Wq, Wout, Wgd, bgd, Wgl, bgl, Wgm, bgm, M1, M2, S1, S2):
    b, s, d = x.shape
    h = M1.shape[0]
    n = b * s
    xf = x.reshape(n, d)

    ncores = 2
    vmem = pltpu.CompilerParams(
        dimension_semantics=("parallel", "arbitrary"),
        vmem_limit_bytes=58 * 1024 * 1024,
    )

    wgt = jnp.concatenate([Wgd, Wgl, Wgm], axis=0).astype(_BF).T
    hbg = 0.5 * jnp.concatenate([bgd, bgl, bgm]).reshape(1, 3 * d)
    wkvt = jnp.concatenate([Wk, Wv], axis=0).astype(_BF).T
    m1t_bf = M1.astype(_BF).T
    m2t_bf = M2.astype(_BF).T
    m2_bf = M2.astype(_BF)

    tn = min(512, n // ncores)
    nb = n // (ncores * tn)
    gate_sums, g1p, g2p = pl.pallas_call(
        _gradg_body,
        grid=(ncores, nb),
        in_specs=[
            pl.BlockSpec((tn, d), lambda i, j: (i * nb + j, 0)),
            pl.BlockSpec(memory_space=pl.ANY),
            pl.BlockSpec(memory_space=pl.ANY),
            pl.BlockSpec(memory_space=pl.ANY),
            pl.BlockSpec(memory_space=pl.ANY),
            pl.BlockSpec(memory_space=pl.ANY),
            pl.BlockSpec((1, 3 * d), lambda i, j: (0, 0)),
        ],
        out_specs=[
            pl.BlockSpec((1, 8, 3 * d), lambda i, j: (i, 0, 0)),
            pl.BlockSpec(memory_space=pl.ANY),
            pl.BlockSpec(memory_space=pl.ANY),
        ],
        out_shape=[
            jax.ShapeDtypeStruct((ncores, 8, 3 * d), _F32),
            jax.ShapeDtypeStruct((ncores, h, d), _F32),
            jax.ShapeDtypeStruct((ncores, d, h), _F32),
        ],
        scratch_shapes=[
            pltpu.VMEM((d, 2 * d), _BF),
            pltpu.VMEM((d, h), _BF),
            pltpu.VMEM((h, d), _BF),
            pltpu.VMEM((d, h), _BF),
            pltpu.VMEM((d, 3 * d), _BF),
            pltpu.VMEM((tn, d), _BF),
            pltpu.VMEM((tn, h), _BF),
            pltpu.VMEM((tn, d), _BF),
            pltpu.VMEM((tn, h), _BF),
            pltpu.VMEM((h, d), _F32),
            pltpu.VMEM((d, h), _F32),
            pltpu.SemaphoreType.DMA((5,)),
        ],
        compiler_params=vmem,
        name="ltm_gradg",
    )(xf, wkvt, m1t_bf, m2t_bf, m2_bf, wgt, hbg)

    alpha = 0.5 + 0.5 * jnp.sum(gate_sums[:, :, :d]) / (n * d)
    theta = 0.5 + 0.5 * jnp.sum(gate_sums[:, :, d:2 * d]) / (n * d)
    eta = 0.5 + 0.5 * jnp.sum(gate_sums[:, :, 2 * d:]) / (n * d)
    scalars = jnp.stack([alpha, theta, eta])

    def _update(m, st, gp, rows, cols):
        rb = 8
        return pl.pallas_call(
            _update_body,
            grid=(rb,),
            in_specs=[
                pl.BlockSpec(memory_space=pltpu.SMEM),
                pl.BlockSpec((rows // rb, cols), lambda i: (i, 0)),
                pl.BlockSpec((rows // rb, cols), lambda i: (i, 0)),
                pl.BlockSpec((rows // rb, cols), lambda i: (i, 0)),
                pl.BlockSpec((rows // rb, cols), lambda i: (i, 0)),
            ],
            out_specs=pl.BlockSpec((rows // rb, cols), lambda i: (i, 0)),
            out_shape=jax.ShapeDtypeStruct((rows, cols), _BF),
            compiler_params=pltpu.CompilerParams(
                dimension_semantics=("parallel",),
            ),
            name="ltm_update",
        )(scalars, m, st, gp[0], gp[1])

    m1n = _update(M1, S1, g1p, h, d)
    m2n = _update(M2, S2, g2p, d, h)

    wqt = Wq.astype(_BF)
    woutt = Wout.astype(_BF)
    m1nt = m1n
    m2nt = m2n
    tnr = min(1024, n // ncores)
    nbr = n // (ncores * tnr)
    out = pl.pallas_call(
        _retr_body,
        grid=(ncores, nbr),
        in_specs=[
            pl.BlockSpec((tnr, d), lambda i, j: (i * nbr + j, 0)),
            pl.BlockSpec((d, d), lambda i, j: (0, 0)),
            pl.BlockSpec((d, d), lambda i, j: (0, 0)),
            pl.BlockSpec(memory_space=pl.ANY),
            pl.BlockSpec(memory_space=pl.ANY),
        ],
        out_specs=pl.BlockSpec((tnr, d), lambda i, j: (i * nbr + j, 0)),
        out_shape=jax.ShapeDtypeStruct((n, d), _F32),
        scratch_shapes=[
            pltpu.VMEM((h, d), _BF),
            pltpu.VMEM((d, h), _BF),
            pltpu.SemaphoreType.DMA((2,)),
        ],
        compiler_params=vmem,
        name="ltm_retrieve",
    )(xf, wqt, woutt, m1nt, m2nt)

    return out.reshape(b, s, d)


# speedup vs baseline: 1.1645x; 1.0477x over previous
import jax
import jax.numpy as jnp
from jax.experimental import pallas as pl
from jax.experimental.pallas import tpu as pltpu

_BF = jnp.bfloat16
_F32 = jnp.float32
_F8 = jnp.float8_e4m3fn
_TN = (((0,), (0,)), ((), ()))
_NT = (((1,), (1,)), ((), ()))


def _gradg_body(x_ref, wkvt_hbm, m1t_hbm, m2t_hbm, m2_hbm, wgt_hbm, hbg_ref,
                gs_ref, g1_hbm, g2_hbm,
                wkvt, m1t, m2t, m2, wgt, ka, aa, ra, dha,
                g1_ref, g2_ref, sems):
    i = pl.program_id(0)
    j = pl.program_id(1)
    nb = pl.num_programs(1)
    d = x_ref.shape[1]
    tn = x_ref.shape[0]
    sub = tn // 2

    @pl.when(j == 0)
    def _():
        c0 = pltpu.make_async_copy(wkvt_hbm, wkvt, sems.at[0])
        c1 = pltpu.make_async_copy(m1t_hbm, m1t, sems.at[1])
        c2 = pltpu.make_async_copy(m2t_hbm, m2t, sems.at[2])
        c3 = pltpu.make_async_copy(m2_hbm, m2, sems.at[3])
        c4 = pltpu.make_async_copy(wgt_hbm, wgt, sems.at[4])
        c0.start(); c1.start(); c2.start(); c3.start(); c4.start()
        c0.wait(); c1.wait(); c2.wait(); c3.wait(); c4.wait()
        g1_ref[...] = jnp.zeros_like(g1_ref)
        g2_ref[...] = jnp.zeros_like(g2_ref)
        gs_ref[...] = jnp.zeros_like(gs_ref)

    for p in range(2):
        sl = slice(p * sub, (p + 1) * sub)
        xs = x_ref[sl, :].astype(_BF)
        gg = jnp.dot(xs.astype(_F8), wgt[...], preferred_element_type=_F32)
        t = jnp.tanh(0.5 * gg + hbg_ref[...])
        c = t.shape[1]
        gs_ref[0] += jnp.sum(t.reshape(sub // 8, 8, c), axis=0)

        kv = jnp.dot(xs, wkvt[...], preferred_element_type=_F32)
        k = kv[:, :d].astype(_BF)
        v = kv[:, d:]
        ka[sl, :] = k
        h = jnp.dot(k, m1t[...], preferred_element_type=_F32)
        sig = 0.5 * (1.0 + jnp.tanh(0.5 * h))
        a = h * sig
        a_bf = a.astype(_BF)
        aa[sl, :] = a_bf
        pred = jnp.dot(a_bf, m2t[...], preferred_element_type=_F32)
        r_bf = ((pred - v) * (2.0 / d)).astype(_BF)
        ra[sl, :] = r_bf
        da = jnp.dot(r_bf, m2[...], preferred_element_type=_F32)
        dha[sl, :] = (da * (sig * (1.0 + h * (1.0 - sig)))).astype(_BF)

    g1_ref[...] += jax.lax.dot_general(dha[...], ka[...], _TN,
                                       preferred_element_type=_F32)
    g2_ref[...] += jax.lax.dot_general(ra[...], aa[...], _TN,
                                       preferred_element_type=_F32)

    @pl.when(j == nb - 1)
    def _():
        c5 = pltpu.make_async_copy(g1_ref, g1_hbm.at[i], sems.at[0])
        c6 = pltpu.make_async_copy(g2_ref, g2_hbm.at[i], sems.at[1])
        c5.start(); c6.start()
        c5.wait(); c6.wait()


def _update_body(sc_ref, m_ref, s_ref, ga_ref, gb_ref, out_ref):
    alpha = sc_ref[0]
    theta = sc_ref[1]
    eta = sc_ref[2]
    upd = ((1.0 - alpha) * m_ref[...] + eta * s_ref[...]
           - theta * (ga_ref[...] + gb_ref[...]))
    out_ref[...] = upd.astype(_BF)


def _retr_body(x_ref, wqt_ref, woutt_ref, m1nt_hbm, m2nt_hbm, out_ref,
               m1nt, m2nt, sems):
    j = pl.program_id(1)

    @pl.when(j == 0)
    def _():
        c0 = pltpu.make_async_copy(m1nt_hbm, m1nt, sems.at[0])
        c1 = pltpu.make_async_copy(m2nt_hbm, m2nt, sems.at[1])
        c0.start(); c1.start()
        c0.wait(); c1.wait()

    half = x_ref.shape[0] // 2
    for p in range(2):
        sl = slice(p * half, (p + 1) * half)
        q = jax.lax.dot_general(x_ref[sl, :].astype(_BF), wqt_ref[...], _NT,
                                preferred_element_type=_F32).astype(_BF)
        hq = jax.lax.dot_general(q, m1nt[...], _NT,
                                 preferred_element_type=_F32)
        aq = (hq * (0.5 * (1.0 + jnp.tanh(0.5 * hq)))).astype(_BF)
        retr = jax.lax.dot_general(aq, m2nt[...], _NT,
                                   preferred_element_type=_F32).astype(_BF)
        out_ref[sl, :] = jax.lax.dot_general(retr, woutt_ref[...], _NT,
                                             preferred_element_type=_F32)


def kernel(x, Wk, Wv, Wq, Wout, Wgd, bgd, Wgl, bgl, Wgm, bgm, M1, M2, S1, S2):
    b, s, d = x.shape
    h = M1.shape[0]
    n = b * s
    xf = x.reshape(n, d)

    ncores = 2
    vmem = pltpu.CompilerParams(
        dimension_semantics=("parallel", "arbitrary"),
        vmem_limit_bytes=58 * 1024 * 1024,
    )

    wgt = jnp.concatenate([Wgd, Wgl, Wgm], axis=0).astype(_F8).T
    hbg = 0.5 * jnp.concatenate([bgd, bgl, bgm]).reshape(1, 3 * d)
    wkvt = jnp.concatenate([Wk, Wv], axis=0).astype(_BF).T
    m1t_bf = M1.astype(_BF).T
    m2t_bf = M2.astype(_BF).T
    m2_bf = M2.astype(_BF)

    tn = min(512, n // ncores)
    nb = n // (ncores * tn)
    gate_sums, g1p, g2p = pl.pallas_call(
        _gradg_body,
        grid=(ncores, nb),
        in_specs=[
            pl.BlockSpec((tn, d), lambda i, j: (i * nb + j, 0)),
            pl.BlockSpec(memory_space=pl.ANY),
            pl.BlockSpec(memory_space=pl.ANY),
            pl.BlockSpec(memory_space=pl.ANY),
            pl.BlockSpec(memory_space=pl.ANY),
            pl.BlockSpec(memory_space=pl.ANY),
            pl.BlockSpec((1, 3 * d), lambda i, j: (0, 0)),
        ],
        out_specs=[
            pl.BlockSpec((1, 8, 3 * d), lambda i, j: (i, 0, 0)),
            pl.BlockSpec(memory_space=pl.ANY),
            pl.BlockSpec(memory_space=pl.ANY),
        ],
        out_shape=[
            jax.ShapeDtypeStruct((ncores, 8, 3 * d), _F32),
            jax.ShapeDtypeStruct((ncores, h, d), _F32),
            jax.ShapeDtypeStruct((ncores, d, h), _F32),
        ],
        scratch_shapes=[
            pltpu.VMEM((d, 2 * d), _BF),
            pltpu.VMEM((d, h), _BF),
            pltpu.VMEM((h, d), _BF),
            pltpu.VMEM((d, h), _BF),
            pltpu.VMEM((d, 3 * d), _F8),
            pltpu.VMEM((tn, d), _BF),
            pltpu.VMEM((tn, h), _BF),
            pltpu.VMEM((tn, d), _BF),
            pltpu.VMEM((tn, h), _BF),
            pltpu.VMEM((h, d), _F32),
            pltpu.VMEM((d, h), _F32),
            pltpu.SemaphoreType.DMA((5,)),
        ],
        compiler_params=vmem,
        name="ltm_gradg",
    )(xf, wkvt, m1t_bf, m2t_bf, m2_bf, wgt, hbg)

    alpha = 0.5 + 0.5 * jnp.sum(gate_sums[:, :, :d]) / (n * d)
    theta = 0.5 + 0.5 * jnp.sum(gate_sums[:, :, d:2 * d]) / (n * d)
    eta = 0.5 + 0.5 * jnp.sum(gate_sums[:, :, 2 * d:]) / (n * d)
    scalars = jnp.stack([alpha, theta, eta])

    def _update(m, st, gp, rows, cols):
        rb = 8
        return pl.pallas_call(
            _update_body,
            grid=(rb,),
            in_specs=[
                pl.BlockSpec(memory_space=pltpu.SMEM),
                pl.BlockSpec((rows // rb, cols), lambda i: (i, 0)),
                pl.BlockSpec((rows // rb, cols), lambda i: (i, 0)),
                pl.BlockSpec((rows // rb, cols), lambda i: (i, 0)),
                pl.BlockSpec((rows // rb, cols), lambda i: (i, 0)),
            ],
            out_specs=pl.BlockSpec((rows // rb, cols), lambda i: (i, 0)),
            out_shape=jax.ShapeDtypeStruct((rows, cols), _BF),
            compiler_params=pltpu.CompilerParams(
                dimension_semantics=("parallel",),
            ),
            name="ltm_update",
        )(scalars, m, st, gp[0], gp[1])

    m1n = _update(M1, S1, g1p, h, d)
    m2n = _update(M2, S2, g2p, d, h)

    wqt = Wq.astype(_BF)
    woutt = Wout.astype(_BF)
    m1nt = m1n
    m2nt = m2n
    tnr = min(1024, n // ncores)
    nbr = n // (ncores * tnr)
    out = pl.pallas_call(
        _retr_body,
        grid=(ncores, nbr),
        in_specs=[
            pl.BlockSpec((tnr, d), lambda i, j: (i * nbr + j, 0)),
            pl.BlockSpec((d, d), lambda i, j: (0, 0)),
            pl.BlockSpec((d, d), lambda i, j: (0, 0)),
            pl.BlockSpec(memory_space=pl.ANY),
            pl.BlockSpec(memory_space=pl.ANY),
        ],
        out_specs=pl.BlockSpec((tnr, d), lambda i, j: (i * nbr + j, 0)),
        out_shape=jax.ShapeDtypeStruct((n, d), _F32),
        scratch_shapes=[
            pltpu.VMEM((h, d), _BF),
            pltpu.VMEM((d, h), _BF),
            pltpu.SemaphoreType.DMA((2,)),
        ],
        compiler_params=vmem,
        name="ltm_retrieve",
    )(xf, wqt, woutt, m1nt, m2nt)

    return out.reshape(b, s, d)


# speedup vs baseline: 1.2071x; 1.0365x over previous
import jax
import jax.numpy as jnp
from jax.experimental import pallas as pl
from jax.experimental.pallas import tpu as pltpu

_BF = jnp.bfloat16
_F32 = jnp.float32
_F8 = jnp.float8_e4m3fn
_TN = (((0,), (0,)), ((), ()))
_NT = (((1,), (1,)), ((), ()))


def _gradg_body(x_ref, wkvt_hbm, m1t_hbm, m2t_hbm, m2_hbm, wgt_hbm, hbg_ref,
                gs_ref, g1_hbm, g2_hbm,
                wkvt, m1t, m2t, m2, wgt, m2t8, ka, aa, ra, dha,
                g1_ref, g2_ref, sems):
    i = pl.program_id(0)
    j = pl.program_id(1)
    nb = pl.num_programs(1)
    d = x_ref.shape[1]
    tn = x_ref.shape[0]
    sub = tn // 2

    @pl.when(j == 0)
    def _():
        c0 = pltpu.make_async_copy(wkvt_hbm, wkvt, sems.at[0])
        c1 = pltpu.make_async_copy(m1t_hbm, m1t, sems.at[1])
        c2 = pltpu.make_async_copy(m2t_hbm, m2t, sems.at[2])
        c3 = pltpu.make_async_copy(m2_hbm, m2, sems.at[3])
        c4 = pltpu.make_async_copy(wgt_hbm, wgt, sems.at[4])
        c0.start(); c1.start(); c2.start(); c3.start(); c4.start()
        c0.wait(); c1.wait(); c2.wait(); c3.wait(); c4.wait()
        m2t8[...] = m2t[...].astype(_F8)
        g1_ref[...] = jnp.zeros_like(g1_ref)
        g2_ref[...] = jnp.zeros_like(g2_ref)
        gs_ref[...] = jnp.zeros_like(gs_ref)

    for p in range(2):
        sl = slice(p * sub, (p + 1) * sub)
        xs = x_ref[sl, :].astype(_BF)
        gg = jnp.dot(xs.astype(_F8), wgt[...], preferred_element_type=_F32)
        t = jnp.tanh(0.5 * gg + hbg_ref[...])
        c = t.shape[1]
        gs_ref[0] += jnp.sum(t.reshape(sub // 8, 8, c), axis=0)

        kv = jnp.dot(xs, wkvt[...], preferred_element_type=_F32)
        k = kv[:, :d].astype(_BF)
        v = kv[:, d:]
        ka[sl, :] = k
        h = jnp.dot(k, m1t[...], preferred_element_type=_F32)
        sig = 0.5 * (1.0 + jnp.tanh(0.5 * h))
        a = h * sig
        a_bf = a.astype(_BF)
        aa[sl, :] = a_bf
        pred = jnp.dot(a_bf.astype(_F8), m2t8[...],
                       preferred_element_type=_F32)
        r_bf = ((pred - v) * (2.0 / d)).astype(_BF)
        ra[sl, :] = r_bf
        da = jnp.dot(r_bf, m2[...], preferred_element_type=_F32)
        dha[sl, :] = (da * (sig * (1.0 + h * (1.0 - sig)))).astype(_BF)

    g1_ref[...] += jax.lax.dot_general(dha[...], ka[...], _TN,
                                       preferred_element_type=_F32)
    g2_ref[...] += jax.lax.dot_general(ra[...], aa[...], _TN,
                                       preferred_element_type=_F32)

    @pl.when(j == nb - 1)
    def _():
        c5 = pltpu.make_async_copy(g1_ref, g1_hbm.at[i], sems.at[0])
        c6 = pltpu.make_async_copy(g2_ref, g2_hbm.at[i], sems.at[1])
        c5.start(); c6.start()
        c5.wait(); c6.wait()


def _update_body(sc_ref, m_ref, s_ref, ga_ref, gb_ref, out_ref):
    alpha = sc_ref[0]
    theta = sc_ref[1]
    eta = sc_ref[2]
    upd = ((1.0 - alpha) * m_ref[...] + eta * s_ref[...]
           - theta * (ga_ref[...] + gb_ref[...]))
    out_ref[...] = upd.astype(_BF)


def _retr_body(x_ref, wqt_ref, woutt_ref, m1nt_hbm, m2nt_hbm, out_ref,
               m1nt, m2nt, sems):
    j = pl.program_id(1)

    @pl.when(j == 0)
    def _():
        c0 = pltpu.make_async_copy(m1nt_hbm, m1nt, sems.at[0])
        c1 = pltpu.make_async_copy(m2nt_hbm, m2nt, sems.at[1])
        c0.start(); c1.start()
        c0.wait(); c1.wait()

    half = x_ref.shape[0] // 2
    for p in range(2):
        sl = slice(p * half, (p + 1) * half)
        q = jax.lax.dot_general(x_ref[sl, :].astype(_BF), wqt_ref[...], _NT,
                                preferred_element_type=_F32).astype(_BF)
        hq = jax.lax.dot_general(q, m1nt[...], _NT,
                                 preferred_element_type=_F32)
        aq = (hq * (0.5 * (1.0 + jnp.tanh(0.5 * hq)))).astype(_BF)
        retr = jax.lax.dot_general(aq, m2nt[...], _NT,
                                   preferred_element_type=_F32).astype(_BF)
        out_ref[sl, :] = jax.lax.dot_general(retr, woutt_ref[...], _NT,
                                             preferred_element_type=_F32)


def kernel(x, Wk, Wv, Wq, Wout, Wgd, bgd, Wgl, bgl, Wgm, bgm, M1, M2, S1, S2):
    b, s, d = x.shape
    h = M1.shape[0]
    n = b * s
    xf = x.reshape(n, d)

    ncores = 2
    vmem = pltpu.CompilerParams(
        dimension_semantics=("parallel", "arbitrary"),
        vmem_limit_bytes=58 * 1024 * 1024,
    )

    wgt = jnp.concatenate([Wgd, Wgl, Wgm], axis=0).astype(_F8).T
    hbg = 0.5 * jnp.concatenate([bgd, bgl, bgm]).reshape(1, 3 * d)
    wkvt = jnp.concatenate([Wk, Wv], axis=0).astype(_BF).T
    m1t_bf = M1.astype(_BF).T
    m2t_bf = M2.astype(_BF).T
    m2_bf = M2.astype(_BF)

    tn = min(512, n // ncores)
    nb = n // (ncores * tn)
    gate_sums, g1p, g2p = pl.pallas_call(
        _gradg_body,
        grid=(ncores, nb),
        in_specs=[
            pl.BlockSpec((tn, d), lambda i, j: (i * nb + j, 0)),
            pl.BlockSpec(memory_space=pl.ANY),
            pl.BlockSpec(memory_space=pl.ANY),
            pl.BlockSpec(memory_space=pl.ANY),
            pl.BlockSpec(memory_space=pl.ANY),
            pl.BlockSpec(memory_space=pl.ANY),
            pl.BlockSpec((1, 3 * d), lambda i, j: (0, 0)),
        ],
        out_specs=[
            pl.BlockSpec((1, 8, 3 * d), lambda i, j: (i, 0, 0)),
            pl.BlockSpec(memory_space=pl.ANY),
            pl.BlockSpec(memory_space=pl.ANY),
        ],
        out_shape=[
            jax.ShapeDtypeStruct((ncores, 8, 3 * d), _F32),
            jax.ShapeDtypeStruct((ncores, h, d), _F32),
            jax.ShapeDtypeStruct((ncores, d, h), _F32),
        ],
        scratch_shapes=[
            pltpu.VMEM((d, 2 * d), _BF),
            pltpu.VMEM((d, h), _BF),
            pltpu.VMEM((h, d), _BF),
            pltpu.VMEM((d, h), _BF),
            pltpu.VMEM((d, 3 * d), _F8),
            pltpu.VMEM((h, d), _F8),
            pltpu.VMEM((tn, d), _BF),
            pltpu.VMEM((tn, h), _BF),
            pltpu.VMEM((tn, d), _BF),
            pltpu.VMEM((tn, h), _BF),
            pltpu.VMEM((h, d), _F32),
            pltpu.VMEM((d, h), _F32),
            pltpu.SemaphoreType.DMA((5,)),
        ],
        compiler_params=vmem,
        name="ltm_gradg",
    )(xf, wkvt, m1t_bf, m2t_bf, m2_bf, wgt, hbg)

    alpha = 0.5 + 0.5 * jnp.sum(gate_sums[:, :, :d]) / (n * d)
    theta = 0.5 + 0.5 * jnp.sum(gate_sums[:, :, d:2 * d]) / (n * d)
    eta = 0.5 + 0.5 * jnp.sum(gate_sums[:, :, 2 * d:]) / (n * d)
    scalars = jnp.stack([alpha, theta, eta])

    def _update(m, st, gp, rows, cols):
        rb = 8
        return pl.pallas_call(
            _update_body,
            grid=(rb,),
            in_specs=[
                pl.BlockSpec(memory_space=pltpu.SMEM),
                pl.BlockSpec((rows // rb, cols), lambda i: (i, 0)),
                pl.BlockSpec((rows // rb, cols), lambda i: (i, 0)),
                pl.BlockSpec((rows // rb, cols), lambda i: (i, 0)),
                pl.BlockSpec((rows // rb, cols), lambda i: (i, 0)),
            ],
            out_specs=pl.BlockSpec((rows // rb, cols), lambda i: (i, 0)),
            out_shape=jax.ShapeDtypeStruct((rows, cols), _BF),
            compiler_params=pltpu.CompilerParams(
                dimension_semantics=("parallel",),
            ),
            name="ltm_update",
        )(scalars, m, st, gp[0], gp[1])

    m1n = _update(M1, S1, g1p, h, d)
    m2n = _update(M2, S2, g2p, d, h)

    wqt = Wq.astype(_BF)
    woutt = Wout.astype(_BF)
    m1nt = m1n
    m2nt = m2n
    tnr = min(1024, n // ncores)
    nbr = n // (ncores * tnr)
    out = pl.pallas_call(
        _retr_body,
        grid=(ncores, nbr),
        in_specs=[
            pl.BlockSpec((tnr, d), lambda i, j: (i * nbr + j, 0)),
            pl.BlockSpec((d, d), lambda i, j: (0, 0)),
            pl.BlockSpec((d, d), lambda i, j: (0, 0)),
            pl.BlockSpec(memory_space=pl.ANY),
            pl.BlockSpec(memory_space=pl.ANY),
        ],
        out_specs=pl.BlockSpec((tnr, d), lambda i, j: (i * nbr + j, 0)),
        out_shape=jax.ShapeDtypeStruct((n, d), _F32),
        scratch_shapes=[
            pltpu.VMEM((h, d), _BF),
            pltpu.VMEM((d, h), _BF),
            pltpu.SemaphoreType.DMA((2,)),
        ],
        compiler_params=vmem,
        name="ltm_retrieve",
    )(xf, wqt, woutt, m1nt, m2nt)

    return out.reshape(b, s, d)
